# Initial kernel scaffold; baseline (speedup 1.0000x reference)
import jax
import jax.numpy as jnp
from jax.experimental import pallas as pl


def kernel(encoded_row, encoded_col, W, robot_lot_idx, robot_lot_step, flow, num_lot_type, num_step):
    raise NotImplementedError("write your pallas kernel here")


# speedup vs baseline: 2.4356x; 2.4356x over previous
import functools

import jax
import jax.numpy as jnp
from jax import lax
from jax.experimental import pallas as pl
from jax.experimental.pallas import tpu as pltpu
from jax.experimental.pallas import tpu_sc as plsc

_NC = 2
_NS = 16
_NW = _NC * _NS


def _sc_gather(row_tab, col_tab, flow_tab, lot_t, step_t, nlt16, nst16,
               B, R, C, D):
    S = 2 * B
    CH = S // _NW
    NH = CH // 128
    NI = CH // 16

    mesh = plsc.VectorSubcoreMesh(core_axis_name="c", subcore_axis_name="s")

    @functools.partial(
        pl.kernel,
        mesh=mesh,
        out_type=[
            jax.ShapeDtypeStruct((S, D), jnp.float32),
            jax.ShapeDtypeStruct((S, D), jnp.float32),
            jax.ShapeDtypeStruct((S,), jnp.float32),
            jax.ShapeDtypeStruct((S,), jnp.float32),
        ],
        scratch_types=[
            pltpu.VMEM((CH,), jnp.int32),
            pltpu.VMEM((CH,), jnp.int32),
            pltpu.VMEM((CH,), jnp.int32),
            pltpu.VMEM((CH,), jnp.int32),
            pltpu.VMEM((CH,), jnp.int32),
            pltpu.VMEM((CH,), jnp.int32),
            pltpu.VMEM((CH,), jnp.float32),
            pltpu.VMEM((CH,), jnp.float32),
            pltpu.VMEM((16,), jnp.int32),
            pltpu.VMEM((16,), jnp.int32),
            pltpu.VMEM((CH,), jnp.int32),
            pltpu.VMEM((CH, D), jnp.float32),
            pltpu.VMEM((CH, D), jnp.float32),
            pltpu.SemaphoreType.DMA,
            pltpu.SemaphoreType.DMA,
        ],
    )
    def sc_body(row_hbm, col_hbm, flow_hbm, lot_hbm, step_hbm,
                nlt_hbm, nst_hbm,
                rows_out, cols_out, rmask_out, cmask_out,
                lot_v, step_v, fidx_v, cidx_v, flidx_v, ns_v,
                rmask_v, cmask_v, nlt_v, nst_v, stage_v,
                rows_v, cols_v, sem0, sem1):
        wid = lax.axis_index("s") * _NC + lax.axis_index("c")
        base = wid * CH
        lane = lax.iota(jnp.int32, 16)

        pltpu.sync_copy(lot_hbm.at[pl.ds(base, CH)], lot_v)
        pltpu.sync_copy(step_hbm.at[pl.ds(base, CH)], step_v)
        pltpu.sync_copy(nlt_hbm, nlt_v)
        pltpu.sync_copy(nst_hbm, nst_v)
        nlt = nlt_v[pl.ds(0, 16)]
        nst = nst_v[pl.ds(0, 16)]

        for i in range(NI):
            sl = pl.ds(i * 16, 16)
            s = base + i * 16 + lane
            b = jnp.bitwise_and(s, B - 1)
            lot = lot_v[sl]
            valid = lot <= nlt
            lf = jnp.where(valid, lot, 0)
            fidx_v[sl] = b * R + lf
            rmask_v[sl] = jnp.where(valid, 1.0, 0.0).astype(jnp.float32)
            ns = step_v[sl] + 1
            ns = jnp.where(ns > nst, 0, ns)
            ns_v[sl] = ns
            flidx_v[sl] = (b * 64 + lf) * 32 + ns

        for h in range(NH):
            hs = pl.ds(h * 128, 128)
            pltpu.async_copy(flow_hbm.at[flidx_v.at[hs]], stage_v.at[hs],
                             sem0).wait()

        for i in range(NI):
            sl = pl.ds(i * 16, 16)
            s = base + i * 16 + lane
            b = jnp.bitwise_and(s, B - 1)
            ns = ns_v[sl]
            stage = stage_v[sl]
            live = jnp.logical_and(ns > 0,
                                   jnp.logical_and(stage >= 1, stage <= C))
            cidx_v[sl] = b * C + jnp.where(live, stage - 1, 0)
            cmask_v[sl] = jnp.where(live, 1.0, 0.0).astype(jnp.float32)

        for h in range(NH):
            hs = pl.ds(h * 128, 128)
            cp0 = pltpu.async_copy(row_hbm.at[fidx_v.at[hs]], rows_v.at[hs],
                                   sem0)
            cp1 = pltpu.async_copy(col_hbm.at[cidx_v.at[hs]], cols_v.at[hs],
                                   sem1)
            cp0.wait()
            cp1.wait()

        pltpu.sync_copy(rows_v, rows_out.at[pl.ds(base, CH)])
        pltpu.sync_copy(cols_v, cols_out.at[pl.ds(base, CH)])
        pltpu.sync_copy(rmask_v, rmask_out.at[pl.ds(base, CH)])
        pltpu.sync_copy(cmask_v, cmask_out.at[pl.ds(base, CH)])

    return sc_body(row_tab, col_tab, flow_tab, lot_t, step_t, nlt16, nst16)


def _tc_combine(rows, cols, rmask, cmask, W, B, D):
    BM = 512
    rows2 = rows.reshape(2, B, D)
    cols2 = cols.reshape(2, B, D)
    rm2 = rmask.reshape(2, B, 1)
    cm2 = cmask.reshape(2, B, 1)

    def tc_body(r_ref, c_ref, mr_ref, mc_ref, w_ref, out_ref):
        e0 = r_ref[0] * mr_ref[0] + c_ref[0] * mc_ref[0]
        e1 = r_ref[1] * mr_ref[1] + c_ref[1] * mc_ref[1]
        w = w_ref[...]
        acc = lax.dot_general(e0, w[:, :D], (((1,), (1,)), ((), ())),
                              preferred_element_type=jnp.float32)
        acc = acc + lax.dot_general(e1, w[:, D:], (((1,), (1,)), ((), ())),
                                    preferred_element_type=jnp.float32)
        out_ref[...] = acc

    return pl.pallas_call(
        tc_body,
        grid=(B // BM,),
        in_specs=[
            pl.BlockSpec((2, BM, D), lambda i: (0, i, 0)),
            pl.BlockSpec((2, BM, D), lambda i: (0, i, 0)),
            pl.BlockSpec((2, BM, 1), lambda i: (0, i, 0)),
            pl.BlockSpec((2, BM, 1), lambda i: (0, i, 0)),
            pl.BlockSpec((D, 2 * D), lambda i: (0, 0)),
        ],
        out_specs=pl.BlockSpec((BM, D), lambda i: (i, 0)),
        out_shape=jax.ShapeDtypeStruct((B, D), jnp.float32),
    )(rows2, cols2, rm2, cm2, W)


@jax.jit
def _run(encoded_row, encoded_col, W, robot_lot_idx, robot_lot_step, flow,
         num_lot_type, num_step):
    B, R, D = encoded_row.shape
    C = encoded_col.shape[1]

    row_tab = encoded_row.reshape(B * R, D)
    col_tab = encoded_col.reshape(B * C, D)
    flow_tab = flow.reshape(-1).astype(jnp.int32)
    lot_t = robot_lot_idx.astype(jnp.int32).T.reshape(-1)
    step_t = robot_lot_step.astype(jnp.int32).T.reshape(-1)
    nlt16 = jnp.broadcast_to(jnp.asarray(num_lot_type, jnp.int32), (16,))
    nst16 = jnp.broadcast_to(jnp.asarray(num_step, jnp.int32), (16,))

    rows, cols, rmask, cmask = _sc_gather(
        row_tab, col_tab, flow_tab, lot_t, step_t, nlt16, nst16,
        B, R, C, D)
    return _tc_combine(rows, cols, rmask, cmask, W, B, D)


def kernel(encoded_row, encoded_col, W, robot_lot_idx, robot_lot_step, flow,
           num_lot_type, num_step):
    return _run(encoded_row, encoded_col, W, robot_lot_idx, robot_lot_step,
                flow, num_lot_type, num_step)


# speedup vs baseline: 6.3923x; 2.6246x over previous
import functools

import jax
import jax.numpy as jnp
from jax import lax
from jax.experimental import pallas as pl
from jax.experimental.pallas import tpu as pltpu
from jax.experimental.pallas import tpu_sc as plsc

_NC = 2
_NS = 16
_NW = _NC * _NS


def _sc_gather(row_tab, col_tab, lot_t, step_t, stage_t, nlt16, nst16,
               B, R, C, D):
    S = 2 * B
    CH = S // _NW
    NH = CH // 128
    NI = CH // 16

    mesh = plsc.VectorSubcoreMesh(core_axis_name="c", subcore_axis_name="s")

    @functools.partial(
        pl.kernel,
        mesh=mesh,
        out_type=[
            jax.ShapeDtypeStruct((S, D), jnp.float32),
            jax.ShapeDtypeStruct((S, D), jnp.float32),
            jax.ShapeDtypeStruct((S,), jnp.float32),
            jax.ShapeDtypeStruct((S,), jnp.float32),
        ],
        scratch_types=[
            pltpu.VMEM((CH,), jnp.int32),
            pltpu.VMEM((CH,), jnp.int32),
            pltpu.VMEM((CH,), jnp.int32),
            pltpu.VMEM((CH,), jnp.int32),
            pltpu.VMEM((CH,), jnp.int32),
            pltpu.VMEM((CH,), jnp.float32),
            pltpu.VMEM((CH,), jnp.float32),
            pltpu.VMEM((16,), jnp.int32),
            pltpu.VMEM((16,), jnp.int32),
            pltpu.VMEM((CH, D), jnp.float32),
            pltpu.VMEM((CH, D), jnp.float32),
            pltpu.SemaphoreType.DMA,
            pltpu.SemaphoreType.DMA,
        ],
    )
    def sc_body(row_hbm, col_hbm, lot_hbm, step_hbm, stage_hbm,
                nlt_hbm, nst_hbm,
                rows_out, cols_out, rmask_out, cmask_out,
                lot_v, step_v, stage_sv, fidx_v, cidx_v,
                rmask_v, cmask_v, nlt_v, nst_v,
                rows_v, cols_v, sem0, sem1):
        wid = lax.axis_index("s") * _NC + lax.axis_index("c")
        base = wid * CH
        lane = lax.iota(jnp.int32, 16)

        pltpu.sync_copy(lot_hbm.at[pl.ds(base, CH)], lot_v)
        pltpu.sync_copy(step_hbm.at[pl.ds(base, CH)], step_v)
        pltpu.sync_copy(stage_hbm.at[pl.ds(base, CH)], stage_sv)
        pltpu.sync_copy(nlt_hbm, nlt_v)
        pltpu.sync_copy(nst_hbm, nst_v)
        nlt = nlt_v[pl.ds(0, 16)]
        nst = nst_v[pl.ds(0, 16)]

        for i in range(NI):
            sl = pl.ds(i * 16, 16)
            s = base + i * 16 + lane
            b = jnp.bitwise_and(s, B - 1)
            lot = lot_v[sl]
            valid = lot <= nlt
            lf = jnp.where(valid, lot, 0)
            fidx_v[sl] = b * R + lf
            rmask_v[sl] = jnp.where(valid, 1.0, 0.0).astype(jnp.float32)
            ns = step_v[sl] + 1
            live_step = ns <= nst
            stage = stage_sv[sl]
            live = jnp.logical_and(live_step,
                                   jnp.logical_and(stage >= 1, stage <= C))
            cidx_v[sl] = b * C + jnp.where(live, stage - 1, 0)
            cmask_v[sl] = jnp.where(live, 1.0, 0.0).astype(jnp.float32)

        cps = []
        for h in range(NH):
            hs = pl.ds(h * 128, 128)
            cps.append(pltpu.async_copy(row_hbm.at[fidx_v.at[hs]],
                                        rows_v.at[hs], sem0))
            cps.append(pltpu.async_copy(col_hbm.at[cidx_v.at[hs]],
                                        cols_v.at[hs], sem1))
        for cp in cps:
            cp.wait()

        pltpu.sync_copy(rows_v, rows_out.at[pl.ds(base, CH)])
        pltpu.sync_copy(cols_v, cols_out.at[pl.ds(base, CH)])
        pltpu.sync_copy(rmask_v, rmask_out.at[pl.ds(base, CH)])
        pltpu.sync_copy(cmask_v, cmask_out.at[pl.ds(base, CH)])

    return sc_body(row_tab, col_tab, lot_t, step_t, stage_t, nlt16, nst16)


def _tc_combine(rows, cols, rmask, cmask, W, B, D):
    BM = 512
    rows2 = rows.reshape(2, B, D)
    cols2 = cols.reshape(2, B, D)
    rm2 = rmask.reshape(2, B, 1)
    cm2 = cmask.reshape(2, B, 1)

    def tc_body(r_ref, c_ref, mr_ref, mc_ref, w_ref, out_ref):
        e0 = r_ref[0] * mr_ref[0] + c_ref[0] * mc_ref[0]
        e1 = r_ref[1] * mr_ref[1] + c_ref[1] * mc_ref[1]
        w = w_ref[...]
        acc = lax.dot_general(e0, w[:, :D], (((1,), (1,)), ((), ())),
                              preferred_element_type=jnp.float32)
        acc = acc + lax.dot_general(e1, w[:, D:], (((1,), (1,)), ((), ())),
                                    preferred_element_type=jnp.float32)
        out_ref[...] = acc

    return pl.pallas_call(
        tc_body,
        grid=(B // BM,),
        in_specs=[
            pl.BlockSpec((2, BM, D), lambda i: (0, i, 0)),
            pl.BlockSpec((2, BM, D), lambda i: (0, i, 0)),
            pl.BlockSpec((2, BM, 1), lambda i: (0, i, 0)),
            pl.BlockSpec((2, BM, 1), lambda i: (0, i, 0)),
            pl.BlockSpec((D, 2 * D), lambda i: (0, 0)),
        ],
        out_specs=pl.BlockSpec((BM, D), lambda i: (i, 0)),
        out_shape=jax.ShapeDtypeStruct((B, D), jnp.float32),
    )(rows2, cols2, rm2, cm2, W)


@jax.jit
def _run(encoded_row, encoded_col, W, robot_lot_idx, robot_lot_step, flow,
         num_lot_type, num_step):
    B, R, D = encoded_row.shape
    C = encoded_col.shape[1]

    row_tab = encoded_row.reshape(B * R, D)
    col_tab = encoded_col.reshape(B * C, D)
    lot = robot_lot_idx.astype(jnp.int32)
    step = robot_lot_step.astype(jnp.int32)

    lf = jnp.where(lot <= num_lot_type, lot, 0)
    ns = step + 1
    dns = jnp.where(ns > num_step, 0, ns)
    stage = flow[jnp.arange(B)[:, None], lf, dns].astype(jnp.int32)

    lot_t = lot.T.reshape(-1)
    step_t = step.T.reshape(-1)
    stage_t = stage.T.reshape(-1)
    nlt16 = jnp.broadcast_to(jnp.asarray(num_lot_type, jnp.int32), (16,))
    nst16 = jnp.broadcast_to(jnp.asarray(num_step, jnp.int32), (16,))

    rows, cols, rmask, cmask = _sc_gather(
        row_tab, col_tab, lot_t, step_t, stage_t, nlt16, nst16,
        B, R, C, D)
    return _tc_combine(rows, cols, rmask, cmask, W, B, D)


def kernel(encoded_row, encoded_col, W, robot_lot_idx, robot_lot_step, flow,
           num_lot_type, num_step):
    return _run(encoded_row, encoded_col, W, robot_lot_idx, robot_lot_step,
                flow, num_lot_type, num_step)


# speedup vs baseline: 8.2345x; 1.2882x over previous
import functools

import jax
import jax.numpy as jnp
from jax import lax
from jax.experimental import pallas as pl
from jax.experimental.pallas import tpu as pltpu
from jax.experimental.pallas import tpu_sc as plsc

_NC = 2
_NS = 16
_NW = _NC * _NS


def _sc_gather(row_tab, col_tab, lot_t, step_t, stage_t, nlt16, nst16,
               B, R, C, D):
    S = 2 * B
    CH = S // _NW
    NH = CH // 128
    NI = CH // 16

    mesh = plsc.VectorSubcoreMesh(core_axis_name="c", subcore_axis_name="s")

    @functools.partial(
        pl.kernel,
        mesh=mesh,
        out_type=jax.ShapeDtypeStruct((S, D), jnp.float32),
        scratch_types=[
            pltpu.VMEM((CH,), jnp.int32),
            pltpu.VMEM((CH,), jnp.int32),
            pltpu.VMEM((CH,), jnp.int32),
            pltpu.VMEM((CH,), jnp.int32),
            pltpu.VMEM((CH,), jnp.int32),
            pltpu.VMEM((CH,), jnp.float32),
            pltpu.VMEM((CH,), jnp.float32),
            pltpu.VMEM((16,), jnp.int32),
            pltpu.VMEM((16,), jnp.int32),
            pltpu.VMEM((CH, D), jnp.float32),
            pltpu.VMEM((CH, D), jnp.float32),
            pltpu.SemaphoreType.DMA,
            pltpu.SemaphoreType.DMA,
        ],
    )
    def sc_body(row_hbm, col_hbm, lot_hbm, step_hbm, stage_hbm,
                nlt_hbm, nst_hbm, emb_out,
                lot_v, step_v, stage_sv, fidx_v, cidx_v,
                rmask_v, cmask_v, nlt_v, nst_v,
                rows_v, cols_v, sem0, sem1):
        wid = lax.axis_index("s") * _NC + lax.axis_index("c")
        base = wid * CH
        lane = lax.iota(jnp.int32, 16)

        pltpu.sync_copy(lot_hbm.at[pl.ds(base, CH)], lot_v)
        pltpu.sync_copy(step_hbm.at[pl.ds(base, CH)], step_v)
        pltpu.sync_copy(stage_hbm.at[pl.ds(base, CH)], stage_sv)
        pltpu.sync_copy(nlt_hbm, nlt_v)
        pltpu.sync_copy(nst_hbm, nst_v)
        nlt = nlt_v[pl.ds(0, 16)]
        nst = nst_v[pl.ds(0, 16)]

        for i in range(NI):
            sl = pl.ds(i * 16, 16)
            s = base + i * 16 + lane
            b = jnp.bitwise_and(s, B - 1)
            lot = lot_v[sl]
            valid = lot <= nlt
            lf = jnp.where(valid, lot, 0)
            fidx_v[sl] = b * R + lf
            rmask_v[sl] = jnp.where(valid, 1.0, 0.0).astype(jnp.float32)
            ns = step_v[sl] + 1
            live_step = ns <= nst
            stage = stage_sv[sl]
            live = jnp.logical_and(live_step,
                                   jnp.logical_and(stage >= 1, stage <= C))
            cidx_v[sl] = b * C + jnp.where(live, stage - 1, 0)
            cmask_v[sl] = jnp.where(live, 1.0, 0.0).astype(jnp.float32)

        cps = []
        for h in range(NH):
            hs = pl.ds(h * 128, 128)
            cps.append(pltpu.async_copy(row_hbm.at[fidx_v.at[hs]],
                                        rows_v.at[hs], sem0))
            cps.append(pltpu.async_copy(col_hbm.at[cidx_v.at[hs]],
                                        cols_v.at[hs], sem1))
        for cp in cps:
            cp.wait()

        def group_body(g, carry):
            gs = pl.ds(pl.multiple_of(g * 16, 16), 16)
            mr16 = rmask_v[gs]
            mc16 = cmask_v[gs]
            dnums = lax.GatherDimensionNumbers(
                offset_dims=(), collapsed_slice_dims=(0,),
                start_index_map=(0,))
            for rl in range(16):
                r = g * 16 + rl
                splat = jnp.full((16, 1), rl, jnp.int32)
                mr = lax.gather(mr16, splat, dnums, (1,),
                                mode=lax.GatherScatterMode.PROMISE_IN_BOUNDS)
                mc = lax.gather(mc16, splat, dnums, (1,),
                                mode=lax.GatherScatterMode.PROMISE_IN_BOUNDS)
                for j in range(D // 16):
                    cs = pl.ds(j * 16, 16)
                    rows_v[r, cs] = rows_v[r, cs] * mr + cols_v[r, cs] * mc
            return carry

        lax.fori_loop(0, NI, group_body, 0)

        pltpu.sync_copy(rows_v, emb_out.at[pl.ds(base, CH)])

    return sc_body(row_tab, col_tab, lot_t, step_t, stage_t, nlt16, nst16)


def _tc_combine(emb, W, B, D):
    BM = 512
    emb2 = emb.reshape(2, B, D)

    def tc_body(r_ref, w_ref, out_ref):
        w = w_ref[...]
        acc = lax.dot_general(r_ref[0], w[:, :D], (((1,), (1,)), ((), ())),
                              preferred_element_type=jnp.float32)
        acc = acc + lax.dot_general(r_ref[1], w[:, D:], (((1,), (1,)), ((), ())),
                                    preferred_element_type=jnp.float32)
        out_ref[...] = acc

    return pl.pallas_call(
        tc_body,
        grid=(B // BM,),
        in_specs=[
            pl.BlockSpec((2, BM, D), lambda i: (0, i, 0)),
            pl.BlockSpec((D, 2 * D), lambda i: (0, 0)),
        ],
        out_specs=pl.BlockSpec((BM, D), lambda i: (i, 0)),
        out_shape=jax.ShapeDtypeStruct((B, D), jnp.float32),
    )(emb2, W)


@jax.jit
def _run(encoded_row, encoded_col, W, robot_lot_idx, robot_lot_step, flow,
         num_lot_type, num_step):
    B, R, D = encoded_row.shape
    C = encoded_col.shape[1]

    row_tab = encoded_row.reshape(B * R, D)
    col_tab = encoded_col.reshape(B * C, D)
    lot = robot_lot_idx.astype(jnp.int32)
    step = robot_lot_step.astype(jnp.int32)

    lot_t = lot.T.reshape(-1)
    step_t = step.T.reshape(-1)
    b_t = jnp.bitwise_and(jnp.arange(2 * B, dtype=jnp.int32), B - 1)
    lf_t = jnp.where(lot_t <= num_lot_type, lot_t, 0)
    ns_t = step_t + 1
    dns_t = jnp.where(ns_t > num_step, 0, ns_t)
    stage_t = flow[b_t, lf_t, dns_t].astype(jnp.int32)

    nlt16 = jnp.broadcast_to(jnp.asarray(num_lot_type, jnp.int32), (16,))
    nst16 = jnp.broadcast_to(jnp.asarray(num_step, jnp.int32), (16,))

    emb = _sc_gather(row_tab, col_tab, lot_t, step_t, stage_t, nlt16, nst16,
                     B, R, C, D)
    return _tc_combine(emb, W, B, D)


def kernel(encoded_row, encoded_col, W, robot_lot_idx, robot_lot_step, flow,
           num_lot_type, num_step):
    return _run(encoded_row, encoded_col, W, robot_lot_idx, robot_lot_step,
                flow, num_lot_type, num_step)
